# baseline (speedup 1.0000x reference)
import functools

import jax
import jax.numpy as jnp
from jax import lax
from jax.experimental import pallas as pl
from jax.experimental.pallas import tpu as pltpu
from jax.experimental.pallas import tpu_sc as plsc

B = 4096
D = 768
K = 1024

BT = 2048
CT_R = 512
CT_C = 2048


def _sim_argmax_body(x_ref, c_ref, sim_ref, idx_ref):
    x = x_ref[...]
    c = c_ref[...]
    dots = lax.dot_general(x, c, (((1,), (1,)), ((), ())),
                           preferred_element_type=jnp.float32)
    xn = jnp.sqrt(jnp.sum(x * x, axis=1))
    cn = jnp.sqrt(jnp.sum(c * c, axis=1))
    denom = jnp.maximum(xn[:, None] * cn[None, :], 1e-8)
    sim = dots / denom
    sim_ref[...] = sim
    idx_ref[0, 0, :] = jnp.argmax(sim, axis=1).astype(jnp.int32)


def _cluster_body(idx_r_ref, idx_c_ref, out_ref):
    i = pl.program_id(0)
    j = pl.program_id(1)
    r = idx_r_ref[0, :]
    c = idx_c_ref[0, :]
    eq = r[:, None] == c[None, :]
    rg = i * CT_R + lax.broadcasted_iota(jnp.int32, (CT_R, CT_C), 0)
    cg = j * CT_C + lax.broadcasted_iota(jnp.int32, (CT_R, CT_C), 1)
    out_ref[...] = jnp.where(eq & (rg != cg), 1.0, 0.0)


_NC = 2
_NS = 16
_NW = _NC * _NS
_BPW = B // _NW


@functools.cache
def _make_sc_gather():
    mesh = plsc.VectorSubcoreMesh(core_axis_name="c", subcore_axis_name="s")

    @functools.partial(
        pl.kernel,
        mesh=mesh,
        out_type=jax.ShapeDtypeStruct((B, D), jnp.float32),
        scratch_types=[
            pltpu.VMEM((_BPW,), jnp.int32),
            pltpu.VMEM((_BPW, D), jnp.float32),
            pltpu.SemaphoreType.DMA,
        ],
    )
    def _sc_gather(table_hbm, idx_hbm, out_hbm, idx_v, rows_v, sem):
        wid = lax.axis_index("s") * _NC + lax.axis_index("c")
        base = wid * _BPW
        pltpu.sync_copy(idx_hbm.at[pl.ds(base, _BPW)], idx_v)
        pltpu.async_copy(table_hbm.at[idx_v], rows_v, sem).wait()
        pltpu.sync_copy(rows_v, out_hbm.at[pl.ds(base, _BPW)])

    return _sc_gather


def kernel(datapoints, batch_cos_sim, centroid):
    sim, idx3 = pl.pallas_call(
        _sim_argmax_body,
        grid=(B // BT,),
        in_specs=[
            pl.BlockSpec((BT, D), lambda i: (i, 0)),
            pl.BlockSpec((K, D), lambda i: (0, 0)),
        ],
        out_specs=[
            pl.BlockSpec((BT, K), lambda i: (i, 0)),
            pl.BlockSpec((1, 1, BT), lambda i: (i, 0, 0)),
        ],
        out_shape=[
            jax.ShapeDtypeStruct((B, K), jnp.float32),
            jax.ShapeDtypeStruct((B // BT, 1, BT), jnp.int32),
        ],
    )(datapoints, centroid)
    dp_index = idx3.reshape(B)
    idx2 = idx3.reshape(1, B)

    dp_cluster = pl.pallas_call(
        _cluster_body,
        grid=(B // CT_R, B // CT_C),
        in_specs=[
            pl.BlockSpec((1, CT_R), lambda i, j: (0, i)),
            pl.BlockSpec((1, CT_C), lambda i, j: (0, j)),
        ],
        out_specs=pl.BlockSpec((CT_R, CT_C), lambda i, j: (i, j)),
        out_shape=jax.ShapeDtypeStruct((B, B), jnp.float32),
    )(idx2, idx2)

    dp_centroid = _make_sc_gather()(centroid, dp_index)
    return sim, dp_index, dp_cluster, dp_centroid


# speedup vs baseline: 1.0533x; 1.0533x over previous
import functools

import jax
import jax.numpy as jnp
from jax import lax
from jax.experimental import pallas as pl
from jax.experimental.pallas import tpu as pltpu
from jax.experimental.pallas import tpu_sc as plsc

B = 4096
D = 768
K = 1024

BT = 1024
CT_R = 1024
CT_C = 2048


def _sim_argmax_body(x_ref, c_ref, sim_ref, idx_ref):
    x = x_ref[...]
    c = c_ref[...]
    dots = lax.dot_general(x, c, (((1,), (1,)), ((), ())),
                           preferred_element_type=jnp.float32)
    xn = jnp.sqrt(jnp.sum(x * x, axis=1))
    cn = jnp.sqrt(jnp.sum(c * c, axis=1))
    denom = jnp.maximum(xn[:, None] * cn[None, :], 1e-8)
    sim = dots / denom
    sim_ref[...] = sim
    idx_ref[0, 0, :] = jnp.argmax(sim, axis=1).astype(jnp.int32)


def _cluster_body(idx_r_ref, idx_c_ref, out_ref):
    i = pl.program_id(0)
    j = pl.program_id(1)
    r = idx_r_ref[0, :]
    c = idx_c_ref[0, :]
    eq = r[:, None] == c[None, :]
    rg = i * CT_R + lax.broadcasted_iota(jnp.int32, (CT_R, CT_C), 0)
    cg = j * CT_C + lax.broadcasted_iota(jnp.int32, (CT_R, CT_C), 1)
    out_ref[...] = jnp.where(eq & (rg != cg), 1.0, 0.0)


_NC = 2
_NS = 16
_NW = _NC * _NS
_BPW = B // _NW


@functools.cache
def _make_sc_gather():
    mesh = plsc.VectorSubcoreMesh(core_axis_name="c", subcore_axis_name="s")

    @functools.partial(
        pl.kernel,
        mesh=mesh,
        out_type=jax.ShapeDtypeStruct((B, D), jnp.float32),
        scratch_types=[
            pltpu.VMEM((_BPW,), jnp.int32),
            pltpu.VMEM((_BPW, D), jnp.float32),
            pltpu.SemaphoreType.DMA,
        ],
    )
    def _sc_gather(table_hbm, idx_hbm, out_hbm, idx_v, rows_v, sem):
        wid = lax.axis_index("s") * _NC + lax.axis_index("c")
        base = wid * _BPW
        pltpu.sync_copy(idx_hbm.at[pl.ds(base, _BPW)], idx_v)
        pltpu.async_copy(table_hbm.at[idx_v], rows_v, sem).wait()
        pltpu.sync_copy(rows_v, out_hbm.at[pl.ds(base, _BPW)])

    return _sc_gather


def kernel(datapoints, batch_cos_sim, centroid):
    sim, idx3 = pl.pallas_call(
        _sim_argmax_body,
        grid=(B // BT,),
        in_specs=[
            pl.BlockSpec((BT, D), lambda i: (i, 0)),
            pl.BlockSpec((K, D), lambda i: (0, 0)),
        ],
        out_specs=[
            pl.BlockSpec((BT, K), lambda i: (i, 0)),
            pl.BlockSpec((1, 1, BT), lambda i: (i, 0, 0)),
        ],
        out_shape=[
            jax.ShapeDtypeStruct((B, K), jnp.float32),
            jax.ShapeDtypeStruct((B // BT, 1, BT), jnp.int32),
        ],
    )(datapoints, centroid)
    dp_index = idx3.reshape(B)
    idx2 = idx3.reshape(1, B)

    dp_cluster = pl.pallas_call(
        _cluster_body,
        grid=(B // CT_R, B // CT_C),
        in_specs=[
            pl.BlockSpec((1, CT_R), lambda i, j: (0, i)),
            pl.BlockSpec((1, CT_C), lambda i, j: (0, j)),
        ],
        out_specs=pl.BlockSpec((CT_R, CT_C), lambda i, j: (i, j)),
        out_shape=jax.ShapeDtypeStruct((B, B), jnp.float32),
    )(idx2, idx2)

    dp_centroid = _make_sc_gather()(centroid, dp_index)
    return sim, dp_index, dp_cluster, dp_centroid
